# Initial kernel scaffold; baseline (speedup 1.0000x reference)
import jax
import jax.numpy as jnp
from jax.experimental import pallas as pl


def kernel(x, edge_index, edge_weight, Wt, bt, Wg, bg, Wh, bh):
    raise NotImplementedError("write your pallas kernel here")


# speedup vs baseline: 17.4490x; 17.4490x over previous
import functools

import jax
import jax.numpy as jnp
from jax import lax
from jax.experimental import pallas as pl
from jax.experimental.pallas import tpu as pltpu
from jax.experimental.pallas import tpu_sc as plsc

N = 100000
E = 1600000
H = 32
WIN = 14

BN = 2048
NBLK = 49
NP = BN * NBLK

CB = 1024

KD = 128
CO = 2048
GPC = CO // KD
EP = 16 * 49 * CO
NCHUNK_D = EP // 16 // CO
EPT = EP // 16
EPW = EP // 32
NCHUNK_B = EPW // CB
RPT = NP // 16
ZR = 784

_mesh = plsc.VectorSubcoreMesh(core_axis_name="c", subcore_axis_name="s")
_sc_params = pltpu.CompilerParams(needs_layout_passes=False,
                                  use_tc_tiling_on_sc=False)


def _tc_xw(x2, W1, bt, WgT):
    def body(xr, w1r, btr, wgr, outr):
        h = jnp.dot(xr[...], w1r[...], preferred_element_type=jnp.float32)
        h = jnp.maximum(h + btr[...], 0.0)
        outr[...] = jnp.dot(h, wgr[...], preferred_element_type=jnp.float32)

    return pl.pallas_call(
        body,
        grid=(NBLK,),
        in_specs=[
            pl.BlockSpec((BN, WIN), lambda i: (i, 0)),
            pl.BlockSpec((WIN, H), lambda i: (0, 0)),
            pl.BlockSpec((1, H), lambda i: (0, 0)),
            pl.BlockSpec((H, H), lambda i: (0, 0)),
        ],
        out_specs=pl.BlockSpec((BN, H), lambda i: (i, 0)),
        out_shape=jax.ShapeDtypeStruct((NP, H), jnp.float32),
    )(x2, W1, bt, WgT)


@functools.partial(
    pl.kernel,
    out_type=jax.ShapeDtypeStruct((32 * NP,), jnp.float32),
    mesh=_mesh,
    scratch_types=[
        pltpu.VMEM((NP,), jnp.float32),
        pltpu.VMEM((CB,), jnp.int32),
        pltpu.VMEM((CB,), jnp.float32),
    ],
    compiler_params=_sc_params,
)
def _sc_deg(col, ew, parts, degbuf, colbuf, ewbuf):
    c = lax.axis_index("c")
    s = lax.axis_index("s")
    wid = s * 2 + c

    def zero(i, _):
        degbuf[pl.ds(i * 16, 16)] = jnp.zeros((16,), jnp.float32)
        return 0

    lax.fori_loop(0, NP // 16, zero, 0)

    base = wid * EPW

    def chunk(k, _):
        off = pl.multiple_of(base + k * CB, CB)
        pltpu.sync_copy(col.at[pl.ds(off, CB)], colbuf)
        pltpu.sync_copy(ew.at[pl.ds(off, CB)], ewbuf)

        def grp(j, _):
            cv = colbuf[pl.ds(j * 16, 16)]
            wv = ewbuf[pl.ds(j * 16, 16)]
            plsc.addupdate_scatter(degbuf, [cv], wv)
            return 0

        lax.fori_loop(0, CB // 16, grp, 0)
        return 0

    lax.fori_loop(0, NCHUNK_B, chunk, 0)
    pltpu.sync_copy(degbuf, parts.at[pl.ds(pl.multiple_of(wid * NP, NP), NP)])


def _tc_prep(parts, xw):
    def body(pr, xwr, dinvr, y0r, y1r):
        deg = jnp.sum(pr[...], axis=0, keepdims=True)
        i = pl.program_id(0)
        gi = i * BN + lax.broadcasted_iota(jnp.int32, (1, BN), 1)
        deg = deg + jnp.where(gi < N, 1.0, 0.0)
        dinv = jnp.where(deg > 0, lax.rsqrt(deg), 0.0)
        dinvr[...] = dinv.reshape(1, 1, BN)
        dc = jnp.transpose(dinv)
        y = xwr[...] * dc
        y0r[...] = y[:, :16]
        y1r[...] = y[:, 16:]

    return pl.pallas_call(
        body,
        grid=(NBLK,),
        in_specs=[
            pl.BlockSpec((32, BN), lambda i: (0, i)),
            pl.BlockSpec((BN, H), lambda i: (i, 0)),
        ],
        out_specs=[
            pl.BlockSpec((1, 1, BN), lambda i: (i, 0, 0)),
            pl.BlockSpec((BN, 16), lambda i: (i, 0)),
            pl.BlockSpec((BN, 16), lambda i: (i, 0)),
        ],
        out_shape=[
            jax.ShapeDtypeStruct((NBLK, 1, BN), jnp.float32),
            jax.ShapeDtypeStruct((NP, 16), jnp.float32),
            jax.ShapeDtypeStruct((NP, 16), jnp.float32),
        ],
    )(parts, xw)


@functools.partial(
    pl.kernel,
    out_type=[
        jax.ShapeDtypeStruct((NP, 16), jnp.float32),
        jax.ShapeDtypeStruct((NP, 16), jnp.float32),
    ],
    mesh=_mesh,
    scratch_types=[
        pltpu.VMEM((CO,), jnp.int32),
        pltpu.VMEM((GPC, KD), jnp.int32),
        pltpu.VMEM((CO,), jnp.float32),
        pltpu.VMEM((KD, 16), jnp.float32),
        pltpu.VMEM((ZR, 16), jnp.float32),
        pltpu.VMEM_SHARED((NP, 16), jnp.float32),
    ],
    compiler_params=_sc_params,
)
def _sc_msg(rowp, colp2, ewp, y0, y1, S0, S1, rowbuf, colbuf, ewbuf, g,
            zbuf, agg):
    c = lax.axis_index("c")
    s = lax.axis_index("s")

    def zrow(i, _):
        zbuf[i] = jnp.zeros((16,), jnp.float32)
        return 0

    lax.fori_loop(0, ZR, zrow, 0)
    rbase = s * RPT

    def zcopy(k, _):
        pltpu.sync_copy(zbuf, agg.at[pl.ds(pl.multiple_of(rbase + k * ZR, ZR), ZR)])
        return 0

    lax.fori_loop(0, RPT // ZR, zcopy, 0)
    plsc.subcore_barrier()

    ebase = s * EPT

    def process(y_ref):
        def chunk(k, _):
            off = pl.multiple_of(ebase + k * CO, CO)
            pltpu.sync_copy(rowp.at[pl.ds(off, CO)], rowbuf)
            pltpu.sync_copy(colp2.at[pl.ds(pl.multiple_of(off // KD, GPC), GPC)],
                            colbuf)
            pltpu.sync_copy(ewp.at[pl.ds(off, CO)], ewbuf)
            for j in range(GPC):
                pltpu.sync_copy(y_ref.at[rowbuf.at[pl.ds(j * KD, KD)]], g)

                def rowi(i, _):
                    idx = lax.broadcast(j * KD + i, (16,))
                    wv = plsc.load_gather(ewbuf, [idx])
                    g[i] = g[i] * wv
                    return 0

                lax.fori_loop(0, KD, rowi, 0)
                pltpu.sync_copy(g, agg.at[colbuf.at[j]], add=True)
            return 0

        lax.fori_loop(0, NCHUNK_D, chunk, 0)

    @pl.when(c == 0)
    def _():
        process(y0)

    @pl.when(c == 1)
    def _():
        process(y1)

    plsc.subcore_barrier()

    rb = pl.multiple_of(rbase, RPT)

    @pl.when(c == 0)
    def _():
        pltpu.sync_copy(agg.at[pl.ds(rb, RPT)], S0.at[pl.ds(rb, RPT)])

    @pl.when(c == 1)
    def _():
        pltpu.sync_copy(agg.at[pl.ds(rb, RPT)], S1.at[pl.ds(rb, RPT)])


def _tc_final(S0, S1, xw, dinv3, bg, WhT, bh):
    def body(s0r, s1r, xwr, dinvr, bgr, whr, bhr, outr):
        dinv = dinvr[...].reshape(1, BN)
        dc = jnp.transpose(dinv)
        S = jnp.concatenate([s0r[...], s1r[...]], axis=1)
        agg = dc * S + (dc * dc) * xwr[...]
        h2 = jnp.maximum(agg + bgr[...], 0.0)
        outr[...] = jnp.sum(h2 * whr[...], axis=1, keepdims=True) + bhr[...]

    return pl.pallas_call(
        body,
        grid=(NBLK,),
        in_specs=[
            pl.BlockSpec((BN, 16), lambda i: (i, 0)),
            pl.BlockSpec((BN, 16), lambda i: (i, 0)),
            pl.BlockSpec((BN, H), lambda i: (i, 0)),
            pl.BlockSpec((1, 1, BN), lambda i: (i, 0, 0)),
            pl.BlockSpec((1, H), lambda i: (0, 0)),
            pl.BlockSpec((1, H), lambda i: (0, 0)),
            pl.BlockSpec((1, 1), lambda i: (0, 0)),
        ],
        out_specs=pl.BlockSpec((BN, 1), lambda i: (i, 0)),
        out_shape=jax.ShapeDtypeStruct((NP, 1), jnp.float32),
    )(S0, S1, xw, dinv3, bg, WhT, bh)


def kernel(x, edge_index, edge_weight, Wt, bt, Wg, bg, Wh, bh):
    x2 = jnp.pad(x.reshape(N, WIN), ((0, NP - N), (0, 0)))
    W1 = Wt.reshape(H, WIN).T
    xw = _tc_xw(x2, W1, bt.reshape(1, H), Wg.T)

    rowp = jnp.pad(edge_index[0], (0, EP - E))
    colp = jnp.pad(edge_index[1], (0, EP - E))
    colp2 = colp.reshape(EP // KD, KD)
    ewp = jnp.pad(edge_weight, (0, EP - E))

    parts = _sc_deg(colp, ewp).reshape(32, NP)
    dinv3, y0, y1 = _tc_prep(parts, xw)
    S0, S1 = _sc_msg(rowp, colp2, ewp, y0, y1)

    outp = _tc_final(S0, S1, xw, dinv3, bg.reshape(1, H), Wh.T,
                     bh.reshape(1, 1))
    return outp[:N]
